# baseline (speedup 1.0000x reference)
import jax
import jax.numpy as jnp
from jax import lax
from jax.experimental import pallas as pl
from jax.experimental.pallas import tpu as pltpu

_MXU_ROWS = 256
_TCHUNK = 8


def _lstm_chunk_kernel(x_ref, wih_ref, whh_ref, b_ref, wlin_ref,
                       blin_ref, out_ref, xt_ref, h_ref, c_ref,
                       wih_s_ref, whh_s_ref):
    batch, u, _ = x_ref.shape
    Hp = whh_ref.shape[0]
    wdtype = whh_ref.dtype
    n_grp = max(1, batch // _MXU_ROWS)
    rows = batch // n_grp
    ci = pl.program_id(1)
    n_c = pl.num_programs(1)

    col = jax.lax.broadcasted_iota(jnp.int32, (1, 4 * Hp), 1) < 3 * Hp
    colf = jnp.where(col, 0.5, 1.0)
    b_s = b_ref[...] * colf

    @pl.when(ci == 0)
    def _init():
        wih_s_ref[...] = (wih_ref[...].astype(jnp.float32) * colf).astype(wdtype)
        whh_s_ref[...] = (whh_ref[...].astype(jnp.float32) * colf).astype(wdtype)
        h_ref[...] = jnp.zeros_like(h_ref)
        c_ref[...] = jnp.zeros_like(c_ref)

    wih_s = wih_s_ref[...]
    whh_s = whh_s_ref[...]

    xt_ref[...] = jnp.swapaxes(x_ref[...].astype(wdtype), 0, 1)

    carry = []
    for j in range(n_grp):
        carry += [h_ref[j * rows:(j + 1) * rows, :],
                  c_ref[j * rows:(j + 1) * rows, :]]

    for k in range(u):
        xt = xt_ref[k]
        new = []
        for j in range(n_grp):
            h, c = carry[2 * j], carry[2 * j + 1]
            pre = (jnp.dot(xt[j * rows:(j + 1) * rows], wih_s,
                           preferred_element_type=jnp.float32)
                   + jnp.dot(h.astype(wdtype), whh_s,
                             preferred_element_type=jnp.float32)
                   + b_s)
            sig = jnp.tanh(pre[:, :3 * Hp]) * 0.5 + 0.5
            i_g = sig[:, 0:Hp]
            f_g = sig[:, Hp:2 * Hp]
            o_g = sig[:, 2 * Hp:3 * Hp]
            g_g = jnp.tanh(pre[:, 3 * Hp:])
            c_new = f_g * c + i_g * g_g
            h_new = o_g * jnp.tanh(c_new)
            new += [h_new, c_new]
        carry = new

    for j in range(n_grp):
        h_ref[j * rows:(j + 1) * rows, :] = carry[2 * j]
        c_ref[j * rows:(j + 1) * rows, :] = carry[2 * j + 1]

    @pl.when(ci == n_c - 1)
    def _head():
        for j in range(n_grp):
            out_ref[j * rows:(j + 1) * rows, :] = (
                jnp.dot(carry[2 * j].astype(wlin_ref.dtype), wlin_ref[...],
                        preferred_element_type=jnp.float32) + blin_ref[...])


def _full_spec(arr):
    nd = arr.ndim
    return pl.BlockSpec(arr.shape, lambda n, t: (0,) * nd)


def _pick_batch(n_seq):
    for b in (512, 256, 128, 64, 32, 16, 8):
        if n_seq % b == 0:
            return b
    return n_seq


@jax.jit
def kernel(xs, wih_f, whh_f, bias_f, wlin_f, blin_f):
    n_seq, seq_len, input_size = xs.shape
    Hp = whh_f.shape[0]
    output_size = wlin_f.shape[1]
    B = _pick_batch(n_seq)
    u = _TCHUNK if seq_len % _TCHUNK == 0 else 1

    return pl.pallas_call(
        _lstm_chunk_kernel,
        out_shape=jax.ShapeDtypeStruct((n_seq, output_size), jnp.float32),
        grid=(n_seq // B, seq_len // u),
        in_specs=[
            pl.BlockSpec((B, u, input_size), lambda n, t: (n, t, 0)),
            _full_spec(wih_f),
            _full_spec(whh_f),
            _full_spec(bias_f),
            _full_spec(wlin_f),
            _full_spec(blin_f),
        ],
        out_specs=pl.BlockSpec((B, output_size), lambda n, t: (n, 0)),
        scratch_shapes=[
            pltpu.VMEM((u, B, input_size), whh_f.dtype),
            pltpu.VMEM((B, Hp), jnp.float32),
            pltpu.VMEM((B, Hp), jnp.float32),
            pltpu.VMEM((input_size, 4 * Hp), whh_f.dtype),
            pltpu.VMEM((Hp, 4 * Hp), whh_f.dtype),
        ],
        compiler_params=pltpu.CompilerParams(
            dimension_semantics=("parallel", "arbitrary")),
    )(xs, wih_f, whh_f, bias_f, wlin_f, blin_f)


# speedup vs baseline: 1.0073x; 1.0073x over previous
import jax
import jax.numpy as jnp
from jax import lax
from jax.experimental import pallas as pl
from jax.experimental.pallas import tpu as pltpu

_MXU_ROWS = 256
_TCHUNK = 32


def _lstm_chunk_kernel(x_ref, wih_ref, whh_ref, b_ref, wlin_ref,
                       blin_ref, out_ref, xt_ref, h_ref, c_ref,
                       wih_s_ref, whh_s_ref):
    batch, u, _ = x_ref.shape
    Hp = whh_ref.shape[0]
    wdtype = whh_ref.dtype
    n_grp = max(1, batch // _MXU_ROWS)
    rows = batch // n_grp
    ci = pl.program_id(1)
    n_c = pl.num_programs(1)

    col = jax.lax.broadcasted_iota(jnp.int32, (1, 4 * Hp), 1) < 3 * Hp
    colf = jnp.where(col, 0.5, 1.0)
    b_s = b_ref[...] * colf

    @pl.when(ci == 0)
    def _init():
        wih_s_ref[...] = (wih_ref[...].astype(jnp.float32) * colf).astype(wdtype)
        whh_s_ref[...] = (whh_ref[...].astype(jnp.float32) * colf).astype(wdtype)
        h_ref[...] = jnp.zeros_like(h_ref)
        c_ref[...] = jnp.zeros_like(c_ref)

    wih_s = wih_s_ref[...]
    whh_s = whh_s_ref[...]

    xt_ref[...] = jnp.swapaxes(x_ref[...].astype(wdtype), 0, 1)

    carry = []
    for j in range(n_grp):
        carry += [h_ref[j * rows:(j + 1) * rows, :],
                  c_ref[j * rows:(j + 1) * rows, :]]

    for k in range(u):
        xt = xt_ref[k]
        new = []
        for j in range(n_grp):
            h, c = carry[2 * j], carry[2 * j + 1]
            pre = (jnp.dot(xt[j * rows:(j + 1) * rows], wih_s,
                           preferred_element_type=jnp.float32)
                   + jnp.dot(h.astype(wdtype), whh_s,
                             preferred_element_type=jnp.float32)
                   + b_s)
            sig = jnp.tanh(pre[:, :3 * Hp]) * 0.5 + 0.5
            i_g = sig[:, 0:Hp]
            f_g = sig[:, Hp:2 * Hp]
            o_g = sig[:, 2 * Hp:3 * Hp]
            g_g = jnp.tanh(pre[:, 3 * Hp:])
            c_new = f_g * c + i_g * g_g
            h_new = o_g * jnp.tanh(c_new)
            new += [h_new, c_new]
        carry = new

    for j in range(n_grp):
        h_ref[j * rows:(j + 1) * rows, :] = carry[2 * j]
        c_ref[j * rows:(j + 1) * rows, :] = carry[2 * j + 1]

    @pl.when(ci == n_c - 1)
    def _head():
        for j in range(n_grp):
            out_ref[j * rows:(j + 1) * rows, :] = (
                jnp.dot(carry[2 * j].astype(wlin_ref.dtype), wlin_ref[...],
                        preferred_element_type=jnp.float32) + blin_ref[...])


def _full_spec(arr):
    nd = arr.ndim
    return pl.BlockSpec(arr.shape, lambda n, t: (0,) * nd)


def _pick_batch(n_seq):
    for b in (512, 256, 128, 64, 32, 16, 8):
        if n_seq % b == 0:
            return b
    return n_seq


@jax.jit
def kernel(xs, wih_f, whh_f, bias_f, wlin_f, blin_f):
    n_seq, seq_len, input_size = xs.shape
    Hp = whh_f.shape[0]
    output_size = wlin_f.shape[1]
    B = _pick_batch(n_seq)
    u = _TCHUNK if seq_len % _TCHUNK == 0 else 1

    return pl.pallas_call(
        _lstm_chunk_kernel,
        out_shape=jax.ShapeDtypeStruct((n_seq, output_size), jnp.float32),
        grid=(n_seq // B, seq_len // u),
        in_specs=[
            pl.BlockSpec((B, u, input_size), lambda n, t: (n, t, 0)),
            _full_spec(wih_f),
            _full_spec(whh_f),
            _full_spec(bias_f),
            _full_spec(wlin_f),
            _full_spec(blin_f),
        ],
        out_specs=pl.BlockSpec((B, output_size), lambda n, t: (n, 0)),
        scratch_shapes=[
            pltpu.VMEM((u, B, input_size), whh_f.dtype),
            pltpu.VMEM((B, Hp), jnp.float32),
            pltpu.VMEM((B, Hp), jnp.float32),
            pltpu.VMEM((input_size, 4 * Hp), whh_f.dtype),
            pltpu.VMEM((Hp, 4 * Hp), whh_f.dtype),
        ],
        compiler_params=pltpu.CompilerParams(
            dimension_semantics=("parallel", "arbitrary")),
    )(xs, wih_f, whh_f, bias_f, wlin_f, blin_f)


# speedup vs baseline: 1.0170x; 1.0096x over previous
import jax
import jax.numpy as jnp
from jax import lax
from jax.experimental import pallas as pl
from jax.experimental.pallas import tpu as pltpu

_MXU_ROWS = 256
_TCHUNK = 16


def _lstm_chunk_kernel(x_ref, wih_ref, whh_ref, b_ref, wlin_ref,
                       blin_ref, out_ref, xt_ref, h_ref, c_ref,
                       wih_s_ref, whh_s_ref):
    batch, u, _ = x_ref.shape
    Hp = whh_ref.shape[0]
    wdtype = whh_ref.dtype
    n_grp = max(1, batch // _MXU_ROWS)
    rows = batch // n_grp
    ci = pl.program_id(1)
    n_c = pl.num_programs(1)

    col = jax.lax.broadcasted_iota(jnp.int32, (1, 4 * Hp), 1) < 3 * Hp
    colf = jnp.where(col, 0.5, 1.0)
    b_s = b_ref[...] * colf

    @pl.when(ci == 0)
    def _init():
        wih_s_ref[...] = (wih_ref[...].astype(jnp.float32) * colf).astype(wdtype)
        whh_s_ref[...] = (whh_ref[...].astype(jnp.float32)
                          * (colf * 0.5)).astype(wdtype)
        h_ref[...] = jnp.zeros_like(h_ref)
        c_ref[...] = jnp.zeros_like(c_ref)

    wih_s = wih_s_ref[...]
    whh_s = whh_s_ref[...]

    xt_ref[...] = jnp.swapaxes(x_ref[...].astype(wdtype), 0, 1)

    carry = []
    for j in range(n_grp):
        carry += [h_ref[j * rows:(j + 1) * rows, :],
                  c_ref[j * rows:(j + 1) * rows, :]]

    for k in range(u):
        xt = xt_ref[k]
        new = []
        for j in range(n_grp):
            h, c = carry[2 * j], carry[2 * j + 1]
            pre = (jnp.dot(xt[j * rows:(j + 1) * rows], wih_s,
                           preferred_element_type=jnp.float32)
                   + jnp.dot(h.astype(wdtype), whh_s,
                             preferred_element_type=jnp.float32)
                   + b_s)
            tg = jnp.tanh(pre[:, :3 * Hp])
            t_i = tg[:, 0:Hp]
            t_f = tg[:, Hp:2 * Hp]
            t_o = tg[:, 2 * Hp:3 * Hp]
            g_g = jnp.tanh(pre[:, 3 * Hp:])
            c_new = ((c + g_g) + (t_f * c + t_i * g_g)) * 0.5
            tc = jnp.tanh(c_new)
            h_new = tc + t_o * tc
            new += [h_new, c_new]
        carry = new

    for j in range(n_grp):
        h_ref[j * rows:(j + 1) * rows, :] = carry[2 * j]
        c_ref[j * rows:(j + 1) * rows, :] = carry[2 * j + 1]

    @pl.when(ci == n_c - 1)
    def _head():
        for j in range(n_grp):
            out_ref[j * rows:(j + 1) * rows, :] = (
                jnp.dot((carry[2 * j] * 0.5).astype(wlin_ref.dtype), wlin_ref[...],
                        preferred_element_type=jnp.float32) + blin_ref[...])


def _full_spec(arr):
    nd = arr.ndim
    return pl.BlockSpec(arr.shape, lambda n, t: (0,) * nd)


def _pick_batch(n_seq):
    for b in (512, 256, 128, 64, 32, 16, 8):
        if n_seq % b == 0:
            return b
    return n_seq


@jax.jit
def kernel(xs, wih_f, whh_f, bias_f, wlin_f, blin_f):
    n_seq, seq_len, input_size = xs.shape
    Hp = whh_f.shape[0]
    output_size = wlin_f.shape[1]
    B = _pick_batch(n_seq)
    u = _TCHUNK if seq_len % _TCHUNK == 0 else 1

    return pl.pallas_call(
        _lstm_chunk_kernel,
        out_shape=jax.ShapeDtypeStruct((n_seq, output_size), jnp.float32),
        grid=(n_seq // B, seq_len // u),
        in_specs=[
            pl.BlockSpec((B, u, input_size), lambda n, t: (n, t, 0)),
            _full_spec(wih_f),
            _full_spec(whh_f),
            _full_spec(bias_f),
            _full_spec(wlin_f),
            _full_spec(blin_f),
        ],
        out_specs=pl.BlockSpec((B, output_size), lambda n, t: (n, 0)),
        scratch_shapes=[
            pltpu.VMEM((u, B, input_size), whh_f.dtype),
            pltpu.VMEM((B, Hp), jnp.float32),
            pltpu.VMEM((B, Hp), jnp.float32),
            pltpu.VMEM((input_size, 4 * Hp), whh_f.dtype),
            pltpu.VMEM((Hp, 4 * Hp), whh_f.dtype),
        ],
        compiler_params=pltpu.CompilerParams(
            dimension_semantics=("parallel", "arbitrary")),
    )(xs, wih_f, whh_f, bias_f, wlin_f, blin_f)


# speedup vs baseline: 1.0271x; 1.0100x over previous
import jax
import jax.numpy as jnp
from jax import lax
from jax.experimental import pallas as pl
from jax.experimental.pallas import tpu as pltpu

_MXU_ROWS = 256
_TCHUNK = 16


def _lstm_chunk_kernel(x_ref, wih_ref, whh_ref, b_ref, wlin_ref,
                       blin_ref, out_ref, xt_ref, h_ref, c_ref,
                       wih_s_ref, whh_s_ref):
    batch, u, _ = x_ref.shape
    Hp = whh_ref.shape[0]
    wdtype = whh_ref.dtype
    n_grp = max(1, batch // _MXU_ROWS)
    rows = batch // n_grp
    ci = pl.program_id(1)
    n_c = pl.num_programs(1)

    col = jax.lax.broadcasted_iota(jnp.int32, (1, 4 * Hp), 1) < 3 * Hp
    colf = jnp.where(col, 0.5, 1.0)
    b_s = b_ref[...] * colf

    @pl.when(ci == 0)
    def _init():
        wih_s_ref[...] = (wih_ref[...].astype(jnp.float32) * colf).astype(wdtype)
        whh_s_ref[...] = (whh_ref[...].astype(jnp.float32) * colf).astype(wdtype)
        h_ref[...] = jnp.zeros_like(h_ref)
        c_ref[...] = jnp.zeros_like(c_ref)

    wih_s = wih_s_ref[...]
    whh_s = whh_s_ref[...]

    xt_ref[...] = jnp.swapaxes(x_ref[...].astype(wdtype), 0, 1)

    carry = []
    for j in range(n_grp):
        carry += [h_ref[j * rows:(j + 1) * rows, :],
                  c_ref[j * rows:(j + 1) * rows, :]]

    for k in range(u):
        xt = xt_ref[k]
        new = []
        for j in range(n_grp):
            h, c = carry[2 * j], carry[2 * j + 1]
            pre = (jnp.dot(xt[j * rows:(j + 1) * rows], wih_s,
                           preferred_element_type=jnp.float32)
                   + jnp.dot(h.astype(wdtype), whh_s,
                             preferred_element_type=jnp.float32)
                   + b_s)
            sig = jnp.tanh(pre[:, :3 * Hp]) * 0.5 + 0.5
            i_g = sig[:, 0:Hp]
            f_g = sig[:, Hp:2 * Hp]
            o_g = sig[:, 2 * Hp:3 * Hp]
            g_g = jnp.tanh(pre[:, 3 * Hp:])
            c_new = f_g * c + i_g * g_g
            h_new = o_g * jnp.tanh(c_new)
            new += [h_new, c_new]
        carry = new

    for j in range(n_grp):
        h_ref[j * rows:(j + 1) * rows, :] = carry[2 * j]
        c_ref[j * rows:(j + 1) * rows, :] = carry[2 * j + 1]

    @pl.when(ci == n_c - 1)
    def _head():
        for j in range(n_grp):
            out_ref[j * rows:(j + 1) * rows, :] = (
                jnp.dot(carry[2 * j].astype(wlin_ref.dtype), wlin_ref[...],
                        preferred_element_type=jnp.float32) + blin_ref[...])


def _full_spec(arr):
    nd = arr.ndim
    return pl.BlockSpec(arr.shape, lambda n, t: (0,) * nd)


def _pick_batch(n_seq):
    for b in (512, 256, 128, 64, 32, 16, 8):
        if n_seq % b == 0:
            return b
    return n_seq


@jax.jit
def kernel(xs, wih_f, whh_f, bias_f, wlin_f, blin_f):
    n_seq, seq_len, input_size = xs.shape
    Hp = whh_f.shape[0]
    output_size = wlin_f.shape[1]
    B = _pick_batch(n_seq)
    u = _TCHUNK if seq_len % _TCHUNK == 0 else 1

    return pl.pallas_call(
        _lstm_chunk_kernel,
        out_shape=jax.ShapeDtypeStruct((n_seq, output_size), jnp.float32),
        grid=(n_seq // B, seq_len // u),
        in_specs=[
            pl.BlockSpec((B, u, input_size), lambda n, t: (n, t, 0)),
            _full_spec(wih_f),
            _full_spec(whh_f),
            _full_spec(bias_f),
            _full_spec(wlin_f),
            _full_spec(blin_f),
        ],
        out_specs=pl.BlockSpec((B, output_size), lambda n, t: (n, 0)),
        scratch_shapes=[
            pltpu.VMEM((u, B, input_size), whh_f.dtype),
            pltpu.VMEM((B, Hp), jnp.float32),
            pltpu.VMEM((B, Hp), jnp.float32),
            pltpu.VMEM((input_size, 4 * Hp), whh_f.dtype),
            pltpu.VMEM((Hp, 4 * Hp), whh_f.dtype),
        ],
        compiler_params=pltpu.CompilerParams(
            dimension_semantics=("parallel", "arbitrary")),
    )(xs, wih_f, whh_f, bias_f, wlin_f, blin_f)
